# baseline (speedup 1.0000x reference)
import functools

import jax
import jax.numpy as jnp
from jax import lax
from jax.experimental import pallas as pl
from jax.experimental.pallas import tpu as pltpu
from jax.experimental.pallas import tpu_sc as plsc

B = 4
NUM_EMB = 100000
D = 32
R = 4096
CC = 50

NC = 2
NS = 16
NW = NC * NS
WPB = NW // B
ROWS_W = R // WPB

RC = 32
NCHUNK = ROWS_W // RC


def _emb_body(idx_hbm, tab_hbm, out_hbm, idx_v, rows_v,
              isem0, isem1, gsem, osem0, osem1):
    wid = lax.axis_index("s") * NC + lax.axis_index("c")
    b = wid // WPB
    j = wid % WPB
    r_base = j * ROWS_W
    isems = (isem0, isem1)
    osems = (osem0, osem1)

    def start_idx_load(g, slot):
        r0 = pl.multiple_of(r_base + g * RC, 8)
        pltpu.async_copy(idx_hbm.at[b, pl.ds(r0, RC)], idx_v.at[slot],
                         isems[slot])

    def wait_idx_load(slot):
        pltpu.make_async_copy(
            idx_hbm.at[0, pl.ds(0, RC)], idx_v.at[slot], isems[slot]).wait()

    def wait_store(slot):
        pltpu.make_async_copy(
            rows_v.at[slot], out_hbm.at[0, pl.ds(0, RC)], osems[slot]).wait()

    def do_chunk(g, slot, first, last):
        if not last:
            @pl.when(g + 1 < NCHUNK)
            def _():
                start_idx_load(g + 1, 1 - slot)
        wait_idx_load(slot)
        if not first:
            wait_store(slot)
        cps = [
            pltpu.async_copy(
                tab_hbm.at[b].at[idx_v.at[slot, r]],
                rows_v.at[slot, r],
                gsem,
            )
            for r in range(RC)
        ]
        for cp in cps:
            cp.wait()
        r0 = pl.multiple_of(r_base + g * RC, 8)
        pltpu.async_copy(rows_v.at[slot], out_hbm.at[b, pl.ds(r0, RC)],
                         osems[slot])

    start_idx_load(0, 0)
    do_chunk(0, 0, True, False)
    do_chunk(1, 1, True, False)

    def pair(h, carry):
        g0 = h * 2
        do_chunk(g0, 0, False, False)
        do_chunk(g0 + 1, 1, False, False)
        return carry

    lax.fori_loop(1, NCHUNK // 2, pair, 0)
    wait_store(0)
    wait_store(1)


@jax.jit
def _run(idx, table):
    mesh = plsc.VectorSubcoreMesh(core_axis_name="c", subcore_axis_name="s")
    f = functools.partial(
        pl.kernel,
        mesh=mesh,
        out_type=jax.ShapeDtypeStruct((B, R, CC, D), jnp.float32),
        scratch_types=[
            pltpu.VMEM((2, RC, CC), jnp.int32),
            pltpu.VMEM((2, RC, CC, D), jnp.float32),
            pltpu.SemaphoreType.DMA,
            pltpu.SemaphoreType.DMA,
            pltpu.SemaphoreType.DMA,
            pltpu.SemaphoreType.DMA,
            pltpu.SemaphoreType.DMA,
        ],
        compiler_params=pltpu.CompilerParams(use_tc_tiling_on_sc=False),
    )(_emb_body)
    return f(idx, table)


def kernel(input, weight):
    return _run(input, weight)
